# baseline (speedup 1.0000x reference)
import functools
import math

import jax
import jax.numpy as jnp
from jax import lax
from jax.experimental import pallas as pl
from jax.experimental.pallas import tpu as pltpu
from jax.experimental.pallas import tpu_sc as plsc

_IDX_W = 128
_CHUNK_ROWS = 128
_R = _CHUNK_ROWS // _IDX_W
_PAD_D = 128
_TC_CHUNK = 512


@functools.cache
def _build_tc_prep(V, D, scale):
    grid = (V + _TC_CHUNK - 1) // _TC_CHUNK

    def body(tT_ref, out_ref):
        eye = (
            lax.broadcasted_iota(jnp.int32, (D, D), 0)
            == lax.broadcasted_iota(jnp.int32, (D, D), 1)
        ).astype(jnp.float32)
        out_ref[:, 0:D] = (
            lax.dot_general(
                tT_ref[...],
                eye,
                (((0,), (0,)), ((), ())),
                preferred_element_type=jnp.float32,
            )
            * scale
        )

    return pl.pallas_call(
        body,
        grid=(grid,),
        in_specs=[pl.BlockSpec((D, _TC_CHUNK), lambda j: (0, j))],
        out_specs=pl.BlockSpec((_TC_CHUNK, _PAD_D), lambda j: (j, 0)),
        out_shape=jax.ShapeDtypeStruct((V, _PAD_D), jnp.float32),
    )


@functools.cache
def _build_sc_gather(N, V, D, n_cores, n_subcores):
    nw = n_cores * n_subcores
    rows_per_worker = N // nw
    n_chunks = rows_per_worker // _CHUNK_ROWS
    n_pairs = n_chunks // 2
    mesh = plsc.VectorSubcoreMesh(core_axis_name="c", subcore_axis_name="s")

    @functools.partial(
        pl.kernel,
        mesh=mesh,
        out_type=jax.ShapeDtypeStruct((N, D), jnp.float32),
        scratch_types=[
            pltpu.VMEM((2, _R, _IDX_W), jnp.int32),
            pltpu.VMEM((2, _CHUNK_ROWS, _PAD_D), jnp.float32),
            pltpu.VMEM((2, _CHUNK_ROWS, D), jnp.float32),
            pltpu.SemaphoreType.DMA,
            pltpu.SemaphoreType.DMA,
            pltpu.SemaphoreType.DMA,
            pltpu.SemaphoreType.DMA,
        ],
        compiler_params=pltpu.CompilerParams(use_tc_tiling_on_sc=True),
    )
    def gather_kernel(
        x_hbm, t128_hbm, out_hbm, idx_v, rows_v, outv, g0, g1, o0, o1
    ):
        wid = lax.axis_index("s") * n_cores + lax.axis_index("c")
        chunk0 = wid * n_chunks
        gsems = (g0, g1)
        osems = (o0, o1)

        def fire_chunk(j, b):
            pltpu.sync_copy(x_hbm.at[pl.ds(j * _R, _R)], idx_v.at[b])
            for r in range(_R):
                pltpu.async_copy(
                    t128_hbm.at[idx_v.at[b, r]],
                    rows_v.at[b, pl.ds(r * _IDX_W, _IDX_W)],
                    gsems[b],
                )

        def wait_chunk(b):
            for r in range(_R):
                pltpu.make_async_copy(
                    t128_hbm.at[idx_v.at[b, r]],
                    rows_v.at[b, pl.ds(r * _IDX_W, _IDX_W)],
                    gsems[b],
                ).wait()

        def compact_chunk(b):
            def body(i, c):
                for k in range(D // 16):
                    sl = pl.ds(k * 16, 16)
                    outv[b, i, sl] = rows_v[b, i, sl]
                return c

            lax.fori_loop(0, _CHUNK_ROWS, body, 0)

        def fire_out(j, b):
            pltpu.async_copy(
                outv.at[b],
                out_hbm.at[pl.ds(j * _CHUNK_ROWS, _CHUNK_ROWS)],
                osems[b],
            )

        def drain_out(j, b):
            pltpu.make_async_copy(
                outv.at[b],
                out_hbm.at[pl.ds(j * _CHUNK_ROWS, _CHUNK_ROWS)],
                osems[b],
            ).wait()

        fire_chunk(chunk0, 0)
        fire_chunk(chunk0 + 1, 1)

        def pair_body(jj, carry):
            c0 = chunk0 + 2 * jj
            c1 = c0 + 1
            wait_chunk(0)
            compact_chunk(0)
            fire_out(c0, 0)
            wait_chunk(1)
            compact_chunk(1)
            fire_out(c1, 1)
            drain_out(c0, 0)
            fire_chunk(c0 + 2, 0)
            drain_out(c1, 1)
            fire_chunk(c1 + 2, 1)
            return carry

        lax.fori_loop(0, n_pairs - 1, pair_body, 0)

        l0 = chunk0 + 2 * (n_pairs - 1)
        wait_chunk(0)
        compact_chunk(0)
        fire_out(l0, 0)
        wait_chunk(1)
        compact_chunk(1)
        fire_out(l0 + 1, 1)
        drain_out(l0, 0)
        drain_out(l0 + 1, 1)

    return gather_kernel


def kernel(x, table):
    B, L = x.shape
    V, D = table.shape
    N = B * L
    scale = float(math.sqrt(D))
    info = plsc.get_sparse_core_info()
    t128 = _build_tc_prep(V, D, scale)(table.T)
    fn = _build_sc_gather(N, V, D, info.num_cores, info.num_subcores)
    x2d = x.reshape(N // _IDX_W, _IDX_W)
    out = fn(x2d, t128)
    return out.reshape(B, L, D)


# speedup vs baseline: 1.7777x; 1.7777x over previous
import functools
import math

import jax
import jax.numpy as jnp
from jax import lax
from jax.experimental import pallas as pl
from jax.experimental.pallas import tpu as pltpu
from jax.experimental.pallas import tpu_sc as plsc

_IDX_W = 128
_CHUNK_ROWS = 128
_R = _CHUNK_ROWS // _IDX_W
_PAD_D = 128
_TC_CHUNK = 4096


@functools.cache
def _build_tc_prep(V, D, scale):
    grid = (V + _TC_CHUNK - 1) // _TC_CHUNK

    def body(tT_ref, out_ref):
        eye = (
            lax.broadcasted_iota(jnp.int32, (D, D), 0)
            == lax.broadcasted_iota(jnp.int32, (D, D), 1)
        ).astype(jnp.float32)
        out_ref[:, 0:D] = (
            lax.dot_general(
                tT_ref[...],
                eye,
                (((0,), (0,)), ((), ())),
                precision=lax.Precision.HIGHEST,
                preferred_element_type=jnp.float32,
            )
            * scale
        )

    return pl.pallas_call(
        body,
        grid=(grid,),
        in_specs=[pl.BlockSpec((D, _TC_CHUNK), lambda j: (0, j))],
        out_specs=pl.BlockSpec((_TC_CHUNK, _PAD_D), lambda j: (j, 0)),
        out_shape=jax.ShapeDtypeStruct((V, _PAD_D), jnp.float32),
    )


@functools.cache
def _build_sc_gather(N, V, D, n_cores, n_subcores):
    nw = n_cores * n_subcores
    rows_per_worker = N // nw
    n_chunks = rows_per_worker // _CHUNK_ROWS
    n_pairs = n_chunks // 2
    mesh = plsc.VectorSubcoreMesh(core_axis_name="c", subcore_axis_name="s")

    @functools.partial(
        pl.kernel,
        mesh=mesh,
        out_type=jax.ShapeDtypeStruct((N, D), jnp.float32),
        scratch_types=[
            pltpu.VMEM((2, _R, _IDX_W), jnp.int32),
            pltpu.VMEM((2, _CHUNK_ROWS, _PAD_D), jnp.float32),
            pltpu.VMEM((2, _CHUNK_ROWS, D), jnp.float32),
            pltpu.SemaphoreType.DMA,
            pltpu.SemaphoreType.DMA,
            pltpu.SemaphoreType.DMA,
            pltpu.SemaphoreType.DMA,
        ],
        compiler_params=pltpu.CompilerParams(use_tc_tiling_on_sc=True),
    )
    def gather_kernel(
        x_hbm, t128_hbm, out_hbm, idx_v, rows_v, outv, g0, g1, o0, o1
    ):
        wid = lax.axis_index("s") * n_cores + lax.axis_index("c")
        chunk0 = wid * n_chunks
        gsems = (g0, g1)
        osems = (o0, o1)

        def fire_chunk(j, b):
            pltpu.sync_copy(x_hbm.at[pl.ds(j * _R, _R)], idx_v.at[b])
            for r in range(_R):
                pltpu.async_copy(
                    t128_hbm.at[idx_v.at[b, r]],
                    rows_v.at[b, pl.ds(r * _IDX_W, _IDX_W)],
                    gsems[b],
                )

        def wait_chunk(b):
            for r in range(_R):
                pltpu.make_async_copy(
                    t128_hbm.at[idx_v.at[b, r]],
                    rows_v.at[b, pl.ds(r * _IDX_W, _IDX_W)],
                    gsems[b],
                ).wait()

        def compact_chunk(b):
            def body(i, c):
                for k in range(D // 16):
                    sl = pl.ds(k * 16, 16)
                    outv[b, i, sl] = rows_v[b, i, sl]
                return c

            lax.fori_loop(0, _CHUNK_ROWS, body, 0)

        def fire_out(j, b):
            pltpu.async_copy(
                outv.at[b],
                out_hbm.at[pl.ds(j * _CHUNK_ROWS, _CHUNK_ROWS)],
                osems[b],
            )

        def drain_out(j, b):
            pltpu.make_async_copy(
                outv.at[b],
                out_hbm.at[pl.ds(j * _CHUNK_ROWS, _CHUNK_ROWS)],
                osems[b],
            ).wait()

        fire_chunk(chunk0, 0)
        fire_chunk(chunk0 + 1, 1)

        def pair_body(jj, carry):
            c0 = chunk0 + 2 * jj
            c1 = c0 + 1
            wait_chunk(0)
            compact_chunk(0)
            fire_out(c0, 0)
            wait_chunk(1)
            compact_chunk(1)
            fire_out(c1, 1)
            drain_out(c0, 0)
            fire_chunk(c0 + 2, 0)
            drain_out(c1, 1)
            fire_chunk(c1 + 2, 1)
            return carry

        lax.fori_loop(0, n_pairs - 1, pair_body, 0)

        l0 = chunk0 + 2 * (n_pairs - 1)
        wait_chunk(0)
        compact_chunk(0)
        fire_out(l0, 0)
        wait_chunk(1)
        compact_chunk(1)
        fire_out(l0 + 1, 1)
        drain_out(l0, 0)
        drain_out(l0 + 1, 1)

    return gather_kernel


def kernel(x, table):
    B, L = x.shape
    V, D = table.shape
    N = B * L
    scale = float(math.sqrt(D))
    info = plsc.get_sparse_core_info()
    t128 = _build_tc_prep(V, D, scale)(table.T)
    fn = _build_sc_gather(N, V, D, info.num_cores, info.num_subcores)
    x2d = x.reshape(N // _IDX_W, _IDX_W)
    out = fn(x2d, t128)
    return out.reshape(B, L, D)


# speedup vs baseline: 2.0625x; 1.1602x over previous
import functools
import math

import jax
import jax.numpy as jnp
from jax import lax
from jax.experimental import pallas as pl
from jax.experimental.pallas import tpu as pltpu
from jax.experimental.pallas import tpu_sc as plsc

_IDX_W = 128
_CHUNK_ROWS = 128
_R = _CHUNK_ROWS // _IDX_W
_PAD_D = 128
_TC_CHUNK = 4096


@functools.cache
def _build_tc_prep(V, D, scale):
    grid = (V + _TC_CHUNK - 1) // _TC_CHUNK

    def body(tT_ref, out_ref):
        out_ref[:, 0:D] = tT_ref[...].T * scale

    return pl.pallas_call(
        body,
        grid=(grid,),
        in_specs=[pl.BlockSpec((D, _TC_CHUNK), lambda j: (0, j))],
        out_specs=pl.BlockSpec((_TC_CHUNK, _PAD_D), lambda j: (j, 0)),
        out_shape=jax.ShapeDtypeStruct((V, _PAD_D), jnp.float32),
    )


@functools.cache
def _build_sc_gather(N, V, D, n_cores, n_subcores):
    nw = n_cores * n_subcores
    rows_per_worker = N // nw
    n_chunks = rows_per_worker // _CHUNK_ROWS
    n_pairs = n_chunks // 2
    mesh = plsc.VectorSubcoreMesh(core_axis_name="c", subcore_axis_name="s")

    @functools.partial(
        pl.kernel,
        mesh=mesh,
        out_type=jax.ShapeDtypeStruct((N, D), jnp.float32),
        scratch_types=[
            pltpu.VMEM((2, _R, _IDX_W), jnp.int32),
            pltpu.VMEM((2, _CHUNK_ROWS, _PAD_D), jnp.float32),
            pltpu.VMEM((2, _CHUNK_ROWS, D), jnp.float32),
            pltpu.SemaphoreType.DMA,
            pltpu.SemaphoreType.DMA,
            pltpu.SemaphoreType.DMA,
            pltpu.SemaphoreType.DMA,
        ],
        compiler_params=pltpu.CompilerParams(use_tc_tiling_on_sc=True),
    )
    def gather_kernel(
        x_hbm, t128_hbm, out_hbm, idx_v, rows_v, outv, g0, g1, o0, o1
    ):
        wid = lax.axis_index("s") * n_cores + lax.axis_index("c")
        chunk0 = wid * n_chunks
        gsems = (g0, g1)
        osems = (o0, o1)

        def fire_chunk(j, b):
            pltpu.sync_copy(x_hbm.at[pl.ds(j * _R, _R)], idx_v.at[b])
            for r in range(_R):
                pltpu.async_copy(
                    t128_hbm.at[idx_v.at[b, r]],
                    rows_v.at[b, pl.ds(r * _IDX_W, _IDX_W)],
                    gsems[b],
                )

        def wait_chunk(b):
            for r in range(_R):
                pltpu.make_async_copy(
                    t128_hbm.at[idx_v.at[b, r]],
                    rows_v.at[b, pl.ds(r * _IDX_W, _IDX_W)],
                    gsems[b],
                ).wait()

        def compact_chunk(b):
            def body(i, c):
                for k in range(D // 16):
                    sl = pl.ds(k * 16, 16)
                    outv[b, i, sl] = rows_v[b, i, sl]
                return c

            lax.fori_loop(0, _CHUNK_ROWS, body, 0)

        def fire_out(j, b):
            pltpu.async_copy(
                outv.at[b],
                out_hbm.at[pl.ds(j * _CHUNK_ROWS, _CHUNK_ROWS)],
                osems[b],
            )

        def drain_out(j, b):
            pltpu.make_async_copy(
                outv.at[b],
                out_hbm.at[pl.ds(j * _CHUNK_ROWS, _CHUNK_ROWS)],
                osems[b],
            ).wait()

        fire_chunk(chunk0, 0)
        fire_chunk(chunk0 + 1, 1)

        def pair_body(jj, carry):
            c0 = chunk0 + 2 * jj
            c1 = c0 + 1
            wait_chunk(0)
            compact_chunk(0)
            fire_out(c0, 0)
            wait_chunk(1)
            compact_chunk(1)
            fire_out(c1, 1)
            drain_out(c0, 0)
            fire_chunk(c0 + 2, 0)
            drain_out(c1, 1)
            fire_chunk(c1 + 2, 1)
            return carry

        lax.fori_loop(0, n_pairs - 1, pair_body, 0)

        l0 = chunk0 + 2 * (n_pairs - 1)
        wait_chunk(0)
        compact_chunk(0)
        fire_out(l0, 0)
        wait_chunk(1)
        compact_chunk(1)
        fire_out(l0 + 1, 1)
        drain_out(l0, 0)
        drain_out(l0 + 1, 1)

    return gather_kernel


def kernel(x, table):
    B, L = x.shape
    V, D = table.shape
    N = B * L
    scale = float(math.sqrt(D))
    info = plsc.get_sparse_core_info()
    t128 = _build_tc_prep(V, D, scale)(table.T)
    fn = _build_sc_gather(N, V, D, info.num_cores, info.num_subcores)
    x2d = x.reshape(N // _IDX_W, _IDX_W)
    out = fn(x2d, t128)
    return out.reshape(B, L, D)


# speedup vs baseline: 2.2462x; 1.0891x over previous
import functools
import math

import jax
import jax.numpy as jnp
from jax import lax
from jax.experimental import pallas as pl
from jax.experimental.pallas import tpu as pltpu
from jax.experimental.pallas import tpu_sc as plsc

_IDX_W = 128
_CHUNK_ROWS = 128
_R = _CHUNK_ROWS // _IDX_W
_PAD_D = 128
_TC_CHUNK = 8192


@functools.cache
def _build_tc_prep(V, D, scale):
    grid = (V + _TC_CHUNK - 1) // _TC_CHUNK

    def body(tT_ref, out_ref):
        out_ref[:, 0:D] = tT_ref[...].T * scale

    return pl.pallas_call(
        body,
        grid=(grid,),
        in_specs=[pl.BlockSpec((D, _TC_CHUNK), lambda j: (0, j))],
        out_specs=pl.BlockSpec((_TC_CHUNK, _PAD_D), lambda j: (j, 0)),
        out_shape=jax.ShapeDtypeStruct((V, _PAD_D), jnp.float32),
    )


@functools.cache
def _build_sc_gather(N, V, D, n_cores, n_subcores):
    nw = n_cores * n_subcores
    rows_per_worker = N // nw
    n_chunks = rows_per_worker // _CHUNK_ROWS
    n_pairs = n_chunks // 2
    mesh = plsc.VectorSubcoreMesh(core_axis_name="c", subcore_axis_name="s")

    @functools.partial(
        pl.kernel,
        mesh=mesh,
        out_type=jax.ShapeDtypeStruct((N, D), jnp.float32),
        scratch_types=[
            pltpu.VMEM((2, _R, _IDX_W), jnp.int32),
            pltpu.VMEM((2, _CHUNK_ROWS, _PAD_D), jnp.float32),
            pltpu.VMEM((2, _CHUNK_ROWS, D), jnp.float32),
            pltpu.SemaphoreType.DMA,
            pltpu.SemaphoreType.DMA,
            pltpu.SemaphoreType.DMA,
            pltpu.SemaphoreType.DMA,
        ],
        compiler_params=pltpu.CompilerParams(use_tc_tiling_on_sc=True),
    )
    def gather_kernel(
        x_hbm, t128_hbm, out_hbm, idx_v, rows_v, outv, g0, g1, o0, o1
    ):
        wid = lax.axis_index("s") * n_cores + lax.axis_index("c")
        chunk0 = wid * n_chunks
        gsems = (g0, g1)
        osems = (o0, o1)

        def fire_chunk(j, b):
            pltpu.sync_copy(x_hbm.at[pl.ds(j * _R, _R)], idx_v.at[b])
            for r in range(_R):
                pltpu.async_copy(
                    t128_hbm.at[idx_v.at[b, r]],
                    rows_v.at[b, pl.ds(r * _IDX_W, _IDX_W)],
                    gsems[b],
                )

        def wait_chunk(b):
            for r in range(_R):
                pltpu.make_async_copy(
                    t128_hbm.at[idx_v.at[b, r]],
                    rows_v.at[b, pl.ds(r * _IDX_W, _IDX_W)],
                    gsems[b],
                ).wait()

        def compact_chunk(b):
            def body(i, c):
                for k in range(D // 16):
                    sl = pl.ds(k * 16, 16)
                    outv[b, i, sl] = rows_v[b, i, sl]
                return c

            lax.fori_loop(0, _CHUNK_ROWS, body, 0)

        def fire_out(j, b):
            pltpu.async_copy(
                outv.at[b],
                out_hbm.at[pl.ds(j * _CHUNK_ROWS, _CHUNK_ROWS)],
                osems[b],
            )

        def drain_out(j, b):
            pltpu.make_async_copy(
                outv.at[b],
                out_hbm.at[pl.ds(j * _CHUNK_ROWS, _CHUNK_ROWS)],
                osems[b],
            ).wait()

        fire_chunk(chunk0, 0)
        fire_chunk(chunk0 + 1, 1)

        def pair_body(jj, carry):
            c0 = chunk0 + 2 * jj
            c1 = c0 + 1
            wait_chunk(0)
            compact_chunk(0)
            fire_out(c0, 0)
            wait_chunk(1)
            compact_chunk(1)
            fire_out(c1, 1)
            drain_out(c0, 0)
            fire_chunk(c0 + 2, 0)
            drain_out(c1, 1)
            fire_chunk(c1 + 2, 1)
            return carry

        lax.fori_loop(0, n_pairs - 1, pair_body, 0)

        l0 = chunk0 + 2 * (n_pairs - 1)
        wait_chunk(0)
        compact_chunk(0)
        fire_out(l0, 0)
        wait_chunk(1)
        compact_chunk(1)
        fire_out(l0 + 1, 1)
        drain_out(l0, 0)
        drain_out(l0 + 1, 1)

    return gather_kernel


def kernel(x, table):
    B, L = x.shape
    V, D = table.shape
    N = B * L
    scale = float(math.sqrt(D))
    info = plsc.get_sparse_core_info()
    t128 = _build_tc_prep(V, D, scale)(table.T)
    fn = _build_sc_gather(N, V, D, info.num_cores, info.num_subcores)
    x2d = x.reshape(N // _IDX_W, _IDX_W)
    out = fn(x2d, t128)
    return out.reshape(B, L, D)


# speedup vs baseline: 2.3043x; 1.0259x over previous
import functools
import math

import jax
import jax.numpy as jnp
from jax import lax
from jax.experimental import pallas as pl
from jax.experimental.pallas import tpu as pltpu
from jax.experimental.pallas import tpu_sc as plsc

_IDX_W = 128
_CHUNK_ROWS = 128
_R = _CHUNK_ROWS // _IDX_W
_PAD_D = 128
_TC_CHUNK = 16384


@functools.cache
def _build_tc_prep(V, D, scale):
    grid = (V + _TC_CHUNK - 1) // _TC_CHUNK

    def body(tT_ref, out_ref):
        out_ref[:, 0:D] = tT_ref[...].T * scale

    return pl.pallas_call(
        body,
        grid=(grid,),
        in_specs=[pl.BlockSpec((D, _TC_CHUNK), lambda j: (0, j))],
        out_specs=pl.BlockSpec((_TC_CHUNK, _PAD_D), lambda j: (j, 0)),
        out_shape=jax.ShapeDtypeStruct((V, _PAD_D), jnp.float32),
    )


@functools.cache
def _build_sc_gather(N, V, D, n_cores, n_subcores):
    nw = n_cores * n_subcores
    rows_per_worker = N // nw
    n_chunks = rows_per_worker // _CHUNK_ROWS
    n_pairs = n_chunks // 2
    mesh = plsc.VectorSubcoreMesh(core_axis_name="c", subcore_axis_name="s")

    @functools.partial(
        pl.kernel,
        mesh=mesh,
        out_type=jax.ShapeDtypeStruct((N, D), jnp.float32),
        scratch_types=[
            pltpu.VMEM((2, _R, _IDX_W), jnp.int32),
            pltpu.VMEM((2, _CHUNK_ROWS, _PAD_D), jnp.float32),
            pltpu.VMEM((2, _CHUNK_ROWS, D), jnp.float32),
            pltpu.SemaphoreType.DMA,
            pltpu.SemaphoreType.DMA,
            pltpu.SemaphoreType.DMA,
            pltpu.SemaphoreType.DMA,
        ],
        compiler_params=pltpu.CompilerParams(use_tc_tiling_on_sc=True),
    )
    def gather_kernel(
        x_hbm, t128_hbm, out_hbm, idx_v, rows_v, outv, g0, g1, o0, o1
    ):
        wid = lax.axis_index("s") * n_cores + lax.axis_index("c")
        chunk0 = wid * n_chunks
        gsems = (g0, g1)
        osems = (o0, o1)

        def fire_chunk(j, b):
            pltpu.sync_copy(x_hbm.at[pl.ds(j * _R, _R)], idx_v.at[b])
            for r in range(_R):
                pltpu.async_copy(
                    t128_hbm.at[idx_v.at[b, r]],
                    rows_v.at[b, pl.ds(r * _IDX_W, _IDX_W)],
                    gsems[b],
                )

        def wait_chunk(b):
            for r in range(_R):
                pltpu.make_async_copy(
                    t128_hbm.at[idx_v.at[b, r]],
                    rows_v.at[b, pl.ds(r * _IDX_W, _IDX_W)],
                    gsems[b],
                ).wait()

        def compact_chunk(b):
            def body(i, c):
                for k in range(D // 16):
                    sl = pl.ds(k * 16, 16)
                    outv[b, i, sl] = rows_v[b, i, sl]
                return c

            lax.fori_loop(0, _CHUNK_ROWS, body, 0)

        def fire_out(j, b):
            pltpu.async_copy(
                outv.at[b],
                out_hbm.at[pl.ds(j * _CHUNK_ROWS, _CHUNK_ROWS)],
                osems[b],
            )

        def drain_out(j, b):
            pltpu.make_async_copy(
                outv.at[b],
                out_hbm.at[pl.ds(j * _CHUNK_ROWS, _CHUNK_ROWS)],
                osems[b],
            ).wait()

        fire_chunk(chunk0, 0)
        fire_chunk(chunk0 + 1, 1)

        def pair_body(jj, carry):
            c0 = chunk0 + 2 * jj
            c1 = c0 + 1
            wait_chunk(0)
            compact_chunk(0)
            fire_out(c0, 0)
            wait_chunk(1)
            compact_chunk(1)
            fire_out(c1, 1)
            drain_out(c0, 0)
            fire_chunk(c0 + 2, 0)
            drain_out(c1, 1)
            fire_chunk(c1 + 2, 1)
            return carry

        lax.fori_loop(0, n_pairs - 1, pair_body, 0)

        l0 = chunk0 + 2 * (n_pairs - 1)
        wait_chunk(0)
        compact_chunk(0)
        fire_out(l0, 0)
        wait_chunk(1)
        compact_chunk(1)
        fire_out(l0 + 1, 1)
        drain_out(l0, 0)
        drain_out(l0 + 1, 1)

    return gather_kernel


def kernel(x, table):
    B, L = x.shape
    V, D = table.shape
    N = B * L
    scale = float(math.sqrt(D))
    info = plsc.get_sparse_core_info()
    t128 = _build_tc_prep(V, D, scale)(table.T)
    fn = _build_sc_gather(N, V, D, info.num_cores, info.num_subcores)
    x2d = x.reshape(N // _IDX_W, _IDX_W)
    out = fn(x2d, t128)
    return out.reshape(B, L, D)


# speedup vs baseline: 2.3234x; 1.0083x over previous
import functools
import math

import jax
import jax.numpy as jnp
from jax import lax
from jax.experimental import pallas as pl
from jax.experimental.pallas import tpu as pltpu
from jax.experimental.pallas import tpu_sc as plsc

_IDX_W = 128
_CHUNK_ROWS = 128
_R = _CHUNK_ROWS // _IDX_W
_PAD_D = 128
_TC_CHUNK = 32768


@functools.cache
def _build_tc_prep(V, D, scale):
    grid = (V + _TC_CHUNK - 1) // _TC_CHUNK

    def body(tT_ref, out_ref):
        out_ref[:, 0:D] = tT_ref[...].T * scale

    return pl.pallas_call(
        body,
        grid=(grid,),
        in_specs=[pl.BlockSpec((D, _TC_CHUNK), lambda j: (0, j))],
        out_specs=pl.BlockSpec((_TC_CHUNK, _PAD_D), lambda j: (j, 0)),
        out_shape=jax.ShapeDtypeStruct((V, _PAD_D), jnp.float32),
    )


@functools.cache
def _build_sc_gather(N, V, D, n_cores, n_subcores):
    nw = n_cores * n_subcores
    rows_per_worker = N // nw
    n_chunks = rows_per_worker // _CHUNK_ROWS
    n_pairs = n_chunks // 2
    mesh = plsc.VectorSubcoreMesh(core_axis_name="c", subcore_axis_name="s")

    @functools.partial(
        pl.kernel,
        mesh=mesh,
        out_type=jax.ShapeDtypeStruct((N, D), jnp.float32),
        scratch_types=[
            pltpu.VMEM((2, _R, _IDX_W), jnp.int32),
            pltpu.VMEM((2, _CHUNK_ROWS, _PAD_D), jnp.float32),
            pltpu.VMEM((2, _CHUNK_ROWS, D), jnp.float32),
            pltpu.SemaphoreType.DMA,
            pltpu.SemaphoreType.DMA,
            pltpu.SemaphoreType.DMA,
            pltpu.SemaphoreType.DMA,
        ],
        compiler_params=pltpu.CompilerParams(use_tc_tiling_on_sc=True),
    )
    def gather_kernel(
        x_hbm, t128_hbm, out_hbm, idx_v, rows_v, outv, g0, g1, o0, o1
    ):
        wid = lax.axis_index("s") * n_cores + lax.axis_index("c")
        chunk0 = wid * n_chunks
        gsems = (g0, g1)
        osems = (o0, o1)

        def fire_chunk(j, b):
            pltpu.sync_copy(x_hbm.at[pl.ds(j * _R, _R)], idx_v.at[b])
            for r in range(_R):
                pltpu.async_copy(
                    t128_hbm.at[idx_v.at[b, r]],
                    rows_v.at[b, pl.ds(r * _IDX_W, _IDX_W)],
                    gsems[b],
                )

        def wait_chunk(b):
            for r in range(_R):
                pltpu.make_async_copy(
                    t128_hbm.at[idx_v.at[b, r]],
                    rows_v.at[b, pl.ds(r * _IDX_W, _IDX_W)],
                    gsems[b],
                ).wait()

        def compact_chunk(b):
            def body(i, c):
                for k in range(D // 16):
                    sl = pl.ds(k * 16, 16)
                    outv[b, i, sl] = rows_v[b, i, sl]
                return c

            lax.fori_loop(0, _CHUNK_ROWS, body, 0)

        def fire_out(j, b):
            pltpu.async_copy(
                outv.at[b],
                out_hbm.at[pl.ds(j * _CHUNK_ROWS, _CHUNK_ROWS)],
                osems[b],
            )

        def drain_out(j, b):
            pltpu.make_async_copy(
                outv.at[b],
                out_hbm.at[pl.ds(j * _CHUNK_ROWS, _CHUNK_ROWS)],
                osems[b],
            ).wait()

        fire_chunk(chunk0, 0)
        fire_chunk(chunk0 + 1, 1)

        def pair_body(jj, carry):
            c0 = chunk0 + 2 * jj
            c1 = c0 + 1
            wait_chunk(0)
            compact_chunk(0)
            fire_out(c0, 0)
            wait_chunk(1)
            compact_chunk(1)
            fire_out(c1, 1)
            drain_out(c0, 0)
            fire_chunk(c0 + 2, 0)
            drain_out(c1, 1)
            fire_chunk(c1 + 2, 1)
            return carry

        lax.fori_loop(0, n_pairs - 1, pair_body, 0)

        l0 = chunk0 + 2 * (n_pairs - 1)
        wait_chunk(0)
        compact_chunk(0)
        fire_out(l0, 0)
        wait_chunk(1)
        compact_chunk(1)
        fire_out(l0 + 1, 1)
        drain_out(l0, 0)
        drain_out(l0 + 1, 1)

    return gather_kernel


def kernel(x, table):
    B, L = x.shape
    V, D = table.shape
    N = B * L
    scale = float(math.sqrt(D))
    info = plsc.get_sparse_core_info()
    t128 = _build_tc_prep(V, D, scale)(table.T)
    fn = _build_sc_gather(N, V, D, info.num_cores, info.num_subcores)
    x2d = x.reshape(N // _IDX_W, _IDX_W)
    out = fn(x2d, t128)
    return out.reshape(B, L, D)
